# Initial kernel scaffold; baseline (speedup 1.0000x reference)
import jax
import jax.numpy as jnp
from jax.experimental import pallas as pl


def kernel(registration_pred, registration_gt, coords):
    raise NotImplementedError("write your pallas kernel here")


# speedup vs baseline: 19.0888x; 19.0888x over previous
import functools

import jax
import jax.numpy as jnp
from jax.experimental import pallas as pl

N = 4096
BM = 256
K10 = 10
K5 = 5


def _self_body(src_ref, dstT_ref, gath_full_ref, gath_blk_ref,
               flow_full_ref, flow_blk_ref, curv_ref, idx_ref, sm_ref,
               *, with_smooth):
    r = pl.program_id(0)
    src = src_ref[...]
    dstT = dstT_ref[...]
    mm = jax.lax.dot_general(src, dstT, (((1,), (0,)), ((), ())),
                             preferred_element_type=jnp.float32)
    ssq = jnp.sum(src * src, axis=1, keepdims=True)
    dsq = jnp.sum(dstT * dstT, axis=0, keepdims=True)
    d = -2.0 * mm + ssq + dsq

    iota = jax.lax.broadcasted_iota(jnp.int32, (BM, N), 1)
    lane16 = jax.lax.broadcasted_iota(jnp.int32, (BM, 16), 1)
    work = d
    onehot = jnp.zeros((BM, N), jnp.float32)
    idxacc = jnp.zeros((BM, 16), jnp.int32)
    sm_part = jnp.zeros((1, 1), jnp.float32)
    for t in range(K10):
        mv = jnp.min(work, axis=1, keepdims=True)
        sel = jnp.min(jnp.where(work == mv, iota, N), axis=1,
                      keepdims=True)
        hit = iota == sel
        hitf = hit.astype(jnp.float32)
        onehot = onehot + hitf
        idxacc = jnp.where(lane16 == t, sel, idxacc)
        if with_smooth and t < 9:
            g = jax.lax.dot_general(hitf, flow_full_ref[...],
                                    (((1,), (0,)), ((), ())),
                                    preferred_element_type=jnp.float32)
            diff = g - flow_blk_ref[...]
            nrm = jnp.sqrt(jnp.sum(diff * diff, axis=1, keepdims=True))
            sm_part = sm_part + jnp.sum(nrm, keepdims=True)
        work = jnp.where(hit, jnp.inf, work)

    gsum = jax.lax.dot_general(onehot, gath_full_ref[...],
                               (((1,), (0,)), ((), ())),
                               preferred_element_type=jnp.float32)
    curv_ref[...] = (gsum - 10.0 * gath_blk_ref[...]) / 9.0
    idx_ref[...] = idxacc

    @pl.when(r == 0)
    def _():
        sm_ref[...] = jnp.zeros((1, 1), jnp.float32)
    sm_ref[...] += sm_part


def _self_call(src, dstT, gath, flow, with_smooth):
    grid = (N // BM,)
    body = functools.partial(_self_body, with_smooth=with_smooth)
    return pl.pallas_call(
        body,
        grid=grid,
        in_specs=[
            pl.BlockSpec((BM, 3), lambda r: (r, 0)),
            pl.BlockSpec((3, N), lambda r: (0, 0)),
            pl.BlockSpec((N, 3), lambda r: (0, 0)),
            pl.BlockSpec((BM, 3), lambda r: (r, 0)),
            pl.BlockSpec((N, 3), lambda r: (0, 0)),
            pl.BlockSpec((BM, 3), lambda r: (r, 0)),
        ],
        out_specs=[
            pl.BlockSpec((BM, 3), lambda r: (r, 0)),
            pl.BlockSpec((BM, 16), lambda r: (r, 0)),
            pl.BlockSpec((1, 1), lambda r: (0, 0)),
        ],
        out_shape=[
            jax.ShapeDtypeStruct((N, 3), jnp.float32),
            jax.ShapeDtypeStruct((N, 16), jnp.int32),
            jax.ShapeDtypeStruct((1, 1), jnp.float32),
        ],
    )(src, dstT, gath, gath, flow, flow)


def _cross_body(src_ref, dstT_ref, c2_full_ref, mc_blk_ref,
                dist2_ref, ch_ref, cv_ref):
    r = pl.program_id(0)
    nr = pl.num_programs(0)
    src = src_ref[...]
    dstT = dstT_ref[...]
    mm = jax.lax.dot_general(src, dstT, (((1,), (0,)), ((), ())),
                             preferred_element_type=jnp.float32)
    ssq = jnp.sum(src * src, axis=1, keepdims=True)
    dsq = jnp.sum(dstT * dstT, axis=0, keepdims=True)
    d = -2.0 * mm + ssq + dsq

    colmin = jnp.min(d, axis=0, keepdims=True)

    iota = jax.lax.broadcasted_iota(jnp.int32, (BM, N), 1)
    work = d
    wsum = jnp.zeros((BM, 1), jnp.float32)
    U = jnp.zeros((BM, N), jnp.float32)
    d1_part = jnp.zeros((1, 1), jnp.float32)
    for t in range(K5):
        mv = jnp.min(work, axis=1, keepdims=True)
        sel = jnp.min(jnp.where(work == mv, iota, N), axis=1, keepdims=True)
        hit = iota == sel
        w = 1.0 / (mv + 1e-8)
        wsum = wsum + w
        U = U + hit.astype(jnp.float32) * w
        if t == 0:
            d1_part = jnp.sum(mv, keepdims=True)
        work = jnp.where(hit, jnp.inf, work)

    inter = jax.lax.dot_general(U, c2_full_ref[...],
                                (((1,), (0,)), ((), ())),
                                preferred_element_type=jnp.float32) / wsum
    cdiff = inter - mc_blk_ref[...]
    cv_part = jnp.sum(cdiff * cdiff, keepdims=True)

    @pl.when(r == 0)
    def _():
        dist2_ref[...] = colmin
        ch_ref[...] = jnp.zeros((1, 1), jnp.float32)
        cv_ref[...] = jnp.zeros((1, 1), jnp.float32)

    @pl.when(r > 0)
    def _():
        dist2_ref[...] = jnp.minimum(dist2_ref[...], colmin)

    ch_ref[...] += d1_part
    cv_ref[...] += cv_part

    @pl.when(r == nr - 1)
    def _():
        ch_ref[...] += jnp.sum(dist2_ref[...], keepdims=True)


def _cross_call(src, dstT, c2, mc):
    grid = (N // BM,)
    return pl.pallas_call(
        _cross_body,
        grid=grid,
        in_specs=[
            pl.BlockSpec((BM, 3), lambda r: (r, 0)),
            pl.BlockSpec((3, N), lambda r: (0, 0)),
            pl.BlockSpec((N, 3), lambda r: (0, 0)),
            pl.BlockSpec((BM, 3), lambda r: (r, 0)),
        ],
        out_specs=[
            pl.BlockSpec((1, N), lambda r: (0, 0)),
            pl.BlockSpec((1, 1), lambda r: (0, 0)),
            pl.BlockSpec((1, 1), lambda r: (0, 0)),
        ],
        out_shape=[
            jax.ShapeDtypeStruct((1, N), jnp.float32),
            jax.ShapeDtypeStruct((1, 1), jnp.float32),
            jax.ShapeDtypeStruct((1, 1), jnp.float32),
        ],
    )(src, dstT, c2, mc)


def kernel(registration_pred, registration_gt, coords):
    flow = registration_pred[0]
    pc1 = coords
    pc2 = coords + registration_gt[0]
    warp = pc1 + flow

    pc1T = pc1.T
    pc2T = pc2.T

    c2, _idx22, _sm0 = _self_call(pc2, pc2T, pc2, flow, with_smooth=False)
    mc, _idx11, sm = _self_call(pc1, pc1T, warp, flow, with_smooth=True)
    _dist2, ch, cv = _cross_call(warp, pc2T, c2, mc)

    chamfer = ch[0, 0]
    curv = cv[0, 0]
    smooth = sm[0, 0] / 8.0

    alpha = 0.02
    total = alpha * chamfer + 0.3 * (alpha * curv) + alpha * smooth
    return jnp.reshape(total, (1,))


# speedup vs baseline: 39.4939x; 2.0690x over previous
import functools

import jax
import jax.numpy as jnp
from jax.experimental import pallas as pl

N = 4096
BM = 256
K10 = 10
K5 = 5
INF = float("inf")


def _dist(a_blk, bT):
    mm = jax.lax.dot_general(a_blk, bT, (((1,), (0,)), ((), ())),
                             preferred_element_type=jnp.float32)
    asq = jnp.sum(a_blk * a_blk, axis=1, keepdims=True)
    bsq = jnp.sum(bT * bT, axis=0, keepdims=True)
    return -2.0 * mm + asq + bsq


def _self_body(src_ref, dstT_ref, gath_full_ref, gath_blk_ref,
               flowT_ref, flow_blk_ref, curv_ref, sm_ref, *, with_smooth):
    r = pl.program_id(0)
    d = _dist(src_ref[...], dstT_ref[...])

    work = d
    sm_part = jnp.zeros((1, 1), jnp.float32)
    for t in range(K10):
        mv = jnp.min(work, axis=1, keepdims=True)
        work = jnp.where(work == mv, INF, work)
        if with_smooth and t == 8:
            m9 = work == INF
            dflow = _dist(flow_blk_ref[...], flowT_ref[...])
            nrm = jnp.sqrt(jnp.maximum(dflow, 0.0))
            sm_part = jnp.sum(jnp.where(m9, nrm, 0.0), keepdims=True)

    onehot = (work == INF).astype(jnp.float32)
    gsum = jax.lax.dot_general(onehot, gath_full_ref[...],
                               (((1,), (0,)), ((), ())),
                               preferred_element_type=jnp.float32)
    curv_ref[...] = (gsum - 10.0 * gath_blk_ref[...]) / 9.0

    if with_smooth:
        @pl.when(r == 0)
        def _():
            sm_ref[...] = jnp.zeros((1, 1), jnp.float32)
        sm_ref[...] += sm_part


def _self_call(src, dstT, gath, flowT, flow, with_smooth):
    grid = (N // BM,)
    body = functools.partial(_self_body, with_smooth=with_smooth)
    out_specs = [pl.BlockSpec((BM, 3), lambda r: (r, 0)),
                 pl.BlockSpec((1, 1), lambda r: (0, 0))]
    out_shape = [jax.ShapeDtypeStruct((N, 3), jnp.float32),
                 jax.ShapeDtypeStruct((1, 1), jnp.float32)]
    return pl.pallas_call(
        body,
        grid=grid,
        in_specs=[
            pl.BlockSpec((BM, 3), lambda r: (r, 0)),
            pl.BlockSpec((3, N), lambda r: (0, 0)),
            pl.BlockSpec((N, 3), lambda r: (0, 0)),
            pl.BlockSpec((BM, 3), lambda r: (r, 0)),
            pl.BlockSpec((3, N), lambda r: (0, 0)),
            pl.BlockSpec((BM, 3), lambda r: (r, 0)),
        ],
        out_specs=out_specs,
        out_shape=out_shape,
    )(src, dstT, gath, gath, flowT, flow)


def _cross_body(src_ref, dstT_ref, c2_full_ref, mc_blk_ref,
                dist2_ref, ch_ref, cv_ref):
    r = pl.program_id(0)
    nr = pl.num_programs(0)
    d = _dist(src_ref[...], dstT_ref[...])

    colmin = jnp.min(d, axis=0, keepdims=True)

    work = d
    d1_part = jnp.zeros((1, 1), jnp.float32)
    for t in range(K5):
        mv = jnp.min(work, axis=1, keepdims=True)
        if t == 0:
            d1_part = jnp.sum(mv, keepdims=True)
        work = jnp.where(work == mv, INF, work)

    m5 = work == INF
    U = jnp.where(m5, 1.0 / (d + 1e-8), 0.0)
    wsum = jnp.sum(U, axis=1, keepdims=True)
    inter = jax.lax.dot_general(U, c2_full_ref[...],
                                (((1,), (0,)), ((), ())),
                                preferred_element_type=jnp.float32) / wsum
    cdiff = inter - mc_blk_ref[...]
    cv_part = jnp.sum(cdiff * cdiff, keepdims=True)

    @pl.when(r == 0)
    def _():
        dist2_ref[...] = colmin
        ch_ref[...] = jnp.zeros((1, 1), jnp.float32)
        cv_ref[...] = jnp.zeros((1, 1), jnp.float32)

    @pl.when(r > 0)
    def _():
        dist2_ref[...] = jnp.minimum(dist2_ref[...], colmin)

    ch_ref[...] += d1_part
    cv_ref[...] += cv_part

    @pl.when(r == nr - 1)
    def _():
        ch_ref[...] += jnp.sum(dist2_ref[...], keepdims=True)


def _cross_call(src, dstT, c2, mc):
    grid = (N // BM,)
    return pl.pallas_call(
        _cross_body,
        grid=grid,
        in_specs=[
            pl.BlockSpec((BM, 3), lambda r: (r, 0)),
            pl.BlockSpec((3, N), lambda r: (0, 0)),
            pl.BlockSpec((N, 3), lambda r: (0, 0)),
            pl.BlockSpec((BM, 3), lambda r: (r, 0)),
        ],
        out_specs=[
            pl.BlockSpec((1, N), lambda r: (0, 0)),
            pl.BlockSpec((1, 1), lambda r: (0, 0)),
            pl.BlockSpec((1, 1), lambda r: (0, 0)),
        ],
        out_shape=[
            jax.ShapeDtypeStruct((1, N), jnp.float32),
            jax.ShapeDtypeStruct((1, 1), jnp.float32),
            jax.ShapeDtypeStruct((1, 1), jnp.float32),
        ],
    )(src, dstT, c2, mc)


def kernel(registration_pred, registration_gt, coords):
    flow = registration_pred[0]
    pc1 = coords
    pc2 = coords + registration_gt[0]
    warp = pc1 + flow

    pc1T = pc1.T
    pc2T = pc2.T
    flowT = flow.T

    c2, _ = _self_call(pc2, pc2T, pc2, flowT, flow, with_smooth=False)
    mc, sm = _self_call(pc1, pc1T, warp, flowT, flow, with_smooth=True)
    _dist2, ch, cv = _cross_call(warp, pc2T, c2, mc)

    chamfer = ch[0, 0]
    curv = cv[0, 0]
    smooth = sm[0, 0] / 8.0

    alpha = 0.02
    total = alpha * chamfer + 0.3 * (alpha * curv) + alpha * smooth
    return jnp.reshape(total, (1,))
